# Initial kernel scaffold; baseline (speedup 1.0000x reference)
import functools

import jax
import jax.numpy as jnp
from jax import lax
from jax.experimental import pallas as pl
from jax.experimental.pallas import tpu as pltpu
from jax.experimental.pallas import tpu_sc as plsc

ENC = 168
SEQ = 199
D_EMB = 64
D_NUM = 16


@functools.cache
def _build_sc_kernel(B, V):
    info = plsc.get_sparse_core_info()
    NC, NS = info.num_cores, info.num_subcores
    NW = NC * NS
    assert B % NW == 0
    rows_per_w = B // NW

    mesh = plsc.VectorSubcoreMesh(core_axis_name="c", subcore_axis_name="s")

    @functools.partial(
        pl.kernel,
        mesh=mesh,
        out_type=(
            jax.ShapeDtypeStruct((B, ENC, D_NUM + D_EMB), jnp.float32),
            jax.ShapeDtypeStruct((B, SEQ, D_EMB), jnp.float32),
        ),
        scratch_types=[
            pltpu.VMEM((SEQ,), jnp.int32),
            pltpu.VMEM((SEQ, D_EMB), jnp.float32),
            pltpu.VMEM((ENC, D_NUM), jnp.float32),
            pltpu.SemaphoreType.DMA,
        ],
    )
    def k(num_hbm, cat_hbm, table_hbm, enc_hbm, emb_hbm, idx_v, rows_v, num_v, sem):
        wid = lax.axis_index("s") * NC + lax.axis_index("c")
        base = wid * rows_per_w

        def body(i, carry):
            b = base + i
            pltpu.sync_copy(cat_hbm.at[b], idx_v)
            cp1 = pltpu.async_copy(
                table_hbm.at[idx_v.at[pl.ds(0, 128)]],
                rows_v.at[pl.ds(0, 128)],
                sem,
            )
            cp2 = pltpu.async_copy(
                table_hbm.at[idx_v.at[pl.ds(128, SEQ - 128)]],
                rows_v.at[pl.ds(128, SEQ - 128)],
                sem,
            )
            pltpu.sync_copy(num_hbm.at[b, pl.ds(0, ENC)], num_v)
            cp1.wait()
            cp2.wait()
            pltpu.sync_copy(rows_v, emb_hbm.at[b])
            pltpu.sync_copy(num_v, enc_hbm.at[b, :, pl.ds(0, D_NUM)])
            pltpu.sync_copy(
                rows_v.at[pl.ds(0, ENC)], enc_hbm.at[b, :, pl.ds(D_NUM, D_EMB)]
            )
            return carry

        lax.fori_loop(0, rows_per_w, body, 0)

    return k


def kernel(input_num, input_cat, table):
    B = input_num.shape[0]
    enc, emb = _build_sc_kernel(B, table.shape[0])(
        input_num, input_cat.astype(jnp.int32), table
    )
    return (enc, emb)


# speedup vs baseline: 2.4202x; 2.4202x over previous
import functools

import jax
import jax.numpy as jnp
from jax import lax
from jax.experimental import pallas as pl
from jax.experimental.pallas import tpu as pltpu
from jax.experimental.pallas import tpu_sc as plsc

ENC = 168
SEQ = 199
D_EMB = 64
D_NUM = 16


@functools.cache
def _build_sc_kernel(B, V):
    info = plsc.get_sparse_core_info()
    NC, NS = info.num_cores, info.num_subcores
    NW = NC * NS
    assert B % NW == 0
    rows_per_w = B // NW

    mesh = plsc.VectorSubcoreMesh(core_axis_name="c", subcore_axis_name="s")

    @functools.partial(
        pl.kernel,
        mesh=mesh,
        out_type=(
            jax.ShapeDtypeStruct((B, ENC, D_NUM + D_EMB), jnp.float32),
            jax.ShapeDtypeStruct((B, SEQ, D_EMB), jnp.float32),
        ),
        scratch_types=[
            pltpu.VMEM((SEQ,), jnp.int32),
            pltpu.VMEM((SEQ, D_EMB), jnp.float32),
            pltpu.VMEM((ENC, D_NUM), jnp.float32),
            pltpu.SemaphoreType.DMA,
        ],
        compiler_params=pltpu.CompilerParams(use_tc_tiling_on_sc=False),
    )
    def k(num_hbm, cat_hbm, table_hbm, enc_hbm, emb_hbm, idx_v, rows_v, num_v, sem):
        wid = lax.axis_index("s") * NC + lax.axis_index("c")
        base = wid * rows_per_w

        def body(i, carry):
            b = base + i
            pltpu.sync_copy(cat_hbm.at[b], idx_v)
            cp1 = pltpu.async_copy(
                table_hbm.at[idx_v.at[pl.ds(0, 128)]],
                rows_v.at[pl.ds(0, 128)],
                sem,
            )
            cp2 = pltpu.async_copy(
                table_hbm.at[idx_v.at[pl.ds(128, SEQ - 128)]],
                rows_v.at[pl.ds(128, SEQ - 128)],
                sem,
            )
            pltpu.sync_copy(num_hbm.at[b, pl.ds(0, ENC)], num_v)
            cp1.wait()
            cp2.wait()
            pltpu.sync_copy(rows_v, emb_hbm.at[b])
            pltpu.sync_copy(num_v, enc_hbm.at[b, :, pl.ds(0, D_NUM)])
            pltpu.sync_copy(
                rows_v.at[pl.ds(0, ENC)], enc_hbm.at[b, :, pl.ds(D_NUM, D_EMB)]
            )
            return carry

        lax.fori_loop(0, rows_per_w, body, 0)

    return k


def kernel(input_num, input_cat, table):
    B = input_num.shape[0]
    enc, emb = _build_sc_kernel(B, table.shape[0])(
        input_num, input_cat.astype(jnp.int32), table
    )
    return (enc, emb)


# speedup vs baseline: 2.8738x; 1.1874x over previous
import functools

import jax
import jax.numpy as jnp
from jax import lax
from jax.experimental import pallas as pl
from jax.experimental.pallas import tpu as pltpu
from jax.experimental.pallas import tpu_sc as plsc

ENC = 168
SEQ = 199
D_EMB = 64
D_NUM = 16
LANES = 16


@functools.cache
def _build_sc_kernel(B, V):
    info = plsc.get_sparse_core_info()
    NC, NS = info.num_cores, info.num_subcores
    NW = NC * NS
    assert D_EMB % NW == 0
    n_phase = D_EMB // NW

    mesh = plsc.VectorSubcoreMesh(core_axis_name="c", subcore_axis_name="s")

    @functools.partial(
        pl.kernel,
        mesh=mesh,
        out_type=(
            jax.ShapeDtypeStruct((ENC, D_NUM + D_EMB, B), jnp.float32),
            jax.ShapeDtypeStruct((SEQ, D_EMB, B), jnp.float32),
        ),
        scratch_types=[
            pltpu.VMEM((V,), jnp.float32),
            pltpu.VMEM((B,), jnp.int32),
            pltpu.VMEM((B,), jnp.float32),
            pltpu.VMEM((D_NUM, B // 4), jnp.float32),
            pltpu.SemaphoreType.DMA,
        ],
        compiler_params=pltpu.CompilerParams(needs_layout_passes=False),
    )
    def k(num_hbm, cat_hbm, tab_hbm, enc_hbm, emb_hbm, trow_v, idx_v, slab_v,
          num_v, sem):
        wid = lax.axis_index("s") * NC + lax.axis_index("c")

        for phase in range(n_phase):
            d = phase * NW + wid
            pltpu.sync_copy(tab_hbm.at[d], trow_v)

            def t_body(t, carry):
                pltpu.sync_copy(cat_hbm.at[t], idx_v)

                def g_body(kk, carry2):
                    vals = plsc.load_gather(
                        trow_v, [idx_v[pl.ds(kk * LANES, LANES)]]
                    )
                    slab_v[pl.ds(kk * LANES, LANES)] = vals
                    return carry2

                lax.fori_loop(0, B // LANES, g_body, 0, unroll=8)
                pltpu.sync_copy(slab_v, emb_hbm.at[t, d])

                @pl.when(t < ENC)
                def _():
                    pltpu.sync_copy(slab_v, enc_hbm.at[t, D_NUM + d])

                return carry

            lax.fori_loop(0, SEQ, t_body, 0)

        def n_body(i, carry):
            t = i * NW + wid

            @pl.when(t < ENC)
            def _():
                for half in range(4):
                    pltpu.sync_copy(
                        num_hbm.at[t, :, pl.ds(half * (B // 4), B // 4)], num_v
                    )
                    pltpu.sync_copy(
                        num_v, enc_hbm.at[t, pl.ds(0, D_NUM),
                                          pl.ds(half * (B // 4), B // 4)]
                    )
            return carry

        lax.fori_loop(0, (ENC + NW - 1) // NW, n_body, 0)

    return k


def kernel(input_num, input_cat, table):
    B = input_num.shape[0]
    num_t = jnp.transpose(input_num, (1, 2, 0))
    idx_t = jnp.transpose(input_cat.astype(jnp.int32), (1, 0))
    tab_t = jnp.transpose(table, (1, 0))
    enc_t, emb_t = _build_sc_kernel(B, table.shape[0])(num_t, idx_t, tab_t)
    enc = jnp.transpose(enc_t, (2, 0, 1))
    emb = jnp.transpose(emb_t, (2, 0, 1))
    return (enc, emb)


# speedup vs baseline: 4.1776x; 1.4537x over previous
import functools

import jax
import jax.numpy as jnp
from jax import lax
from jax.experimental import pallas as pl
from jax.experimental.pallas import tpu as pltpu
from jax.experimental.pallas import tpu_sc as plsc

ENC = 168
SEQ = 199
D_EMB = 64
D_NUM = 16
LANES = 16


@functools.cache
def _build_sc_kernel(B, V):
    info = plsc.get_sparse_core_info()
    NC, NS = info.num_cores, info.num_subcores
    NW = NC * NS
    assert D_EMB % NW == 0
    n_phase = D_EMB // NW
    n_pieces = ENC
    half_w = B // 2

    mesh = plsc.VectorSubcoreMesh(core_axis_name="c", subcore_axis_name="s")

    @functools.partial(
        pl.kernel,
        mesh=mesh,
        out_type=(
            jax.ShapeDtypeStruct((ENC, D_NUM + D_EMB, B), jnp.float32),
            jax.ShapeDtypeStruct((SEQ, D_EMB, B), jnp.float32),
        ),
        scratch_types=[
            pltpu.VMEM((V,), jnp.float32),
            [pltpu.VMEM((B,), jnp.int32) for _ in range(2)],
            [pltpu.VMEM((B,), jnp.float32) for _ in range(2)],
            [pltpu.VMEM((B // 2,), jnp.float32) for _ in range(4)],
            [pltpu.SemaphoreType.DMA for _ in range(2)],
            [pltpu.SemaphoreType.DMA for _ in range(2)],
            [pltpu.SemaphoreType.DMA for _ in range(4)],
            [pltpu.SemaphoreType.DMA for _ in range(4)],
        ],
        compiler_params=pltpu.CompilerParams(needs_layout_passes=False),
    )
    def k(num_hbm, cat_hbm, tab_hbm, enc_hbm, emb_hbm,
          trow_v, idxb, slab, numb, sem_i, sem_o, sem_ni, sem_no):
        wid = lax.axis_index("s") * NC + lax.axis_index("c")

        def gather_t(ib, sl):
            def g_body(kk, c):
                sl[pl.ds(kk * LANES, LANES)] = plsc.load_gather(
                    trow_v, [ib[pl.ds(kk * LANES, LANES)]]
                )
                return c

            lax.fori_loop(0, B // LANES, g_body, 0, unroll=8)

        for phase in range(n_phase):
            d = phase * NW + wid
            pltpu.sync_copy(tab_hbm.at[d], trow_v)
            pltpu.async_copy(cat_hbm.at[0], idxb[0], sem_i[0])
            pltpu.async_copy(cat_hbm.at[1], idxb[1], sem_i[1])

            def step(p, t, tp):
                pltpu.make_async_copy(cat_hbm.at[0], idxb[p], sem_i[p]).wait()

                @pl.when(tp >= 1)
                def _():
                    pltpu.make_async_copy(
                        slab[p], emb_hbm.at[0, d], sem_o[p]
                    ).wait()

                    @pl.when(t - 2 < ENC)
                    def _():
                        pltpu.make_async_copy(
                            slab[p], enc_hbm.at[0, D_NUM + d], sem_o[p]
                        ).wait()

                gather_t(idxb[p], slab[p])

                @pl.when(t + 2 < SEQ)
                def _():
                    pltpu.async_copy(cat_hbm.at[t + 2], idxb[p], sem_i[p])

                pltpu.async_copy(slab[p], emb_hbm.at[t, d], sem_o[p])

                @pl.when(t < ENC)
                def _():
                    pltpu.async_copy(slab[p], enc_hbm.at[t, D_NUM + d], sem_o[p])

            def pair(tp, c):
                step(0, 2 * tp, tp)

                @pl.when(2 * tp + 1 < SEQ)
                def _():
                    step(1, 2 * tp + 1, tp)

                return c

            lax.fori_loop(0, (SEQ + 1) // 2, pair, 0)
            pltpu.make_async_copy(slab[0], emb_hbm.at[0, d], sem_o[0]).wait()
            pltpu.make_async_copy(slab[1], emb_hbm.at[0, d], sem_o[1]).wait()

        dd = wid >> 1
        off = pl.multiple_of((wid & 1) * half_w, half_w)

        def piece_src(i):
            return num_hbm.at[i, dd, pl.ds(off, half_w)]

        def piece_dst(i):
            return enc_hbm.at[i, dd, pl.ds(off, half_w)]

        pltpu.async_copy(piece_src(0), numb[0], sem_ni[0])
        pltpu.async_copy(piece_src(1), numb[1], sem_ni[1])

        def nquad(j, c):
            for q in range(4):
                i = 4 * j + q
                b2 = (q + 2) % 4
                pltpu.make_async_copy(
                    num_hbm.at[0, 0, pl.ds(0, half_w)], numb[q], sem_ni[q]
                ).wait()
                pltpu.async_copy(numb[q], piece_dst(i), sem_no[q])

                def drain_and_prefetch():
                    pltpu.make_async_copy(
                        numb[b2], enc_hbm.at[0, 0, pl.ds(0, half_w)], sem_no[b2]
                    ).wait()

                    @pl.when(i + 2 < n_pieces)
                    def _():
                        pltpu.async_copy(piece_src(i + 2), numb[b2], sem_ni[b2])

                if q < 2:
                    @pl.when(j >= 1)
                    def _():
                        drain_and_prefetch()

                    @pl.when(j == 0)
                    def _():
                        pltpu.async_copy(piece_src(i + 2), numb[b2], sem_ni[b2])
                else:
                    drain_and_prefetch()

            return c

        lax.fori_loop(0, n_pieces // 4, nquad, 0)
        pltpu.make_async_copy(numb[2], enc_hbm.at[0, 0, pl.ds(0, half_w)], sem_no[2]).wait()
        pltpu.make_async_copy(numb[3], enc_hbm.at[0, 0, pl.ds(0, half_w)], sem_no[3]).wait()

    return k


def kernel(input_num, input_cat, table):
    B = input_num.shape[0]
    num_t = jnp.transpose(input_num, (1, 2, 0))
    idx_t = jnp.transpose(input_cat.astype(jnp.int32), (1, 0))
    tab_t = jnp.transpose(table, (1, 0))
    enc_t, emb_t = _build_sc_kernel(B, table.shape[0])(num_t, idx_t, tab_t)
    enc = jnp.transpose(enc_t, (2, 0, 1))
    emb = jnp.transpose(emb_t, (2, 0, 1))
    return (enc, emb)


# speedup vs baseline: 12.1886x; 2.9176x over previous
import functools

import jax
import jax.numpy as jnp
from jax import lax
from jax.experimental import pallas as pl
from jax.experimental.pallas import tpu as pltpu
from jax.experimental.pallas import tpu_sc as plsc

ENC = 168
SEQ = 199
D_EMB = 64
D_NUM = 16
LANES = 16
TCHUNK = 8


@functools.cache
def _build_sc_kernel(B, V):
    info = plsc.get_sparse_core_info()
    NC, NS = info.num_cores, info.num_subcores
    NW = NC * NS
    assert D_EMB % NW == 0
    n_phase = D_EMB // NW
    n_chunk = (SEQ + TCHUNK - 1) // TCHUNK
    n_pieces = ENC
    half_w = B // 2

    mesh = plsc.VectorSubcoreMesh(core_axis_name="c", subcore_axis_name="s")

    @functools.partial(
        pl.kernel,
        mesh=mesh,
        out_type=(
            jax.ShapeDtypeStruct((ENC, D_NUM + D_EMB, B), jnp.float32),
            jax.ShapeDtypeStruct((SEQ, D_EMB, B), jnp.float32),
        ),
        scratch_types=[
            pltpu.VMEM((V,), jnp.float32),
            pltpu.VMEM_SHARED((2, TCHUNK, B), jnp.int32),
            [pltpu.VMEM((B,), jnp.int32) for _ in range(2)],
            [pltpu.VMEM((B,), jnp.float32) for _ in range(2)],
            [pltpu.VMEM((B // 2,), jnp.float32) for _ in range(4)],
            [pltpu.SemaphoreType.DMA for _ in range(2)],
            [pltpu.SemaphoreType.DMA for _ in range(2)],
            [pltpu.SemaphoreType.DMA for _ in range(4)],
            [pltpu.SemaphoreType.DMA for _ in range(4)],
            pltpu.SemaphoreType.DMA,
        ],
        compiler_params=pltpu.CompilerParams(needs_layout_passes=False),
    )
    def k(num_hbm, cat_hbm, tab_hbm, enc_hbm, emb_hbm,
          trow_v, spm_idx, idxb, slab, numb, sem_i, sem_o, sem_ni, sem_no,
          sem_p, ):
        sid = lax.axis_index("s")
        wid = sid * NC + lax.axis_index("c")
        is_prod = sid == 0

        def fetch_chunk(c, buf):
            for i in range(TCHUNK):
                @pl.when(c * TCHUNK + i < SEQ)
                def _():
                    pltpu.async_copy(
                        cat_hbm.at[c * TCHUNK + i], spm_idx.at[buf, i], sem_p
                    )

        def wait_chunk(c):
            for i in range(TCHUNK):
                @pl.when(c * TCHUNK + i < SEQ)
                def _():
                    pltpu.make_async_copy(
                        cat_hbm.at[0], spm_idx.at[0, 0], sem_p
                    ).wait()

        def gather_t(ib, sl):
            U = 16

            def g_body(kk, c):
                base = kk * (U * LANES)
                idxs = [ib[pl.ds(base + u * LANES, LANES)] for u in range(U)]
                vals = [plsc.load_gather(trow_v, [iv]) for iv in idxs]
                for u in range(U):
                    sl[pl.ds(base + u * LANES, LANES)] = vals[u]
                return c

            lax.fori_loop(0, B // (U * LANES), g_body, 0)

        for phase in range(n_phase):
            d = phase * NW + wid
            pltpu.sync_copy(tab_hbm.at[d], trow_v)

            @pl.when(is_prod)
            def _():
                fetch_chunk(0, 0)
                wait_chunk(0)

            plsc.subcore_barrier()

            def step(p, lt, c, cur):
                t = c * TCHUNK + lt
                pltpu.make_async_copy(
                    spm_idx.at[0, 0], idxb[p], sem_i[p]
                ).wait()

                @pl.when((c >= 1) | (lt >= 2))
                def _():
                    pltpu.make_async_copy(
                        slab[p], emb_hbm.at[0, d], sem_o[p]
                    ).wait()

                    @pl.when(t - 2 < ENC)
                    def _():
                        pltpu.make_async_copy(
                            slab[p], enc_hbm.at[0, D_NUM + d], sem_o[p]
                        ).wait()

                gather_t(idxb[p], slab[p])

                if lt + 2 < TCHUNK:
                    @pl.when(t + 2 < SEQ)
                    def _():
                        pltpu.async_copy(
                            spm_idx.at[cur, lt + 2], idxb[p], sem_i[p]
                        )

                pltpu.async_copy(slab[p], emb_hbm.at[t, d], sem_o[p])

                @pl.when(t < ENC)
                def _():
                    pltpu.async_copy(slab[p], enc_hbm.at[t, D_NUM + d], sem_o[p])

            def chunk_body(c, carry):
                cur = lax.rem(c, 2)
                nxt = 1 - cur

                @pl.when(is_prod & (c < n_chunk - 1))
                def _():
                    fetch_chunk(c + 1, nxt)

                pltpu.async_copy(spm_idx.at[cur, 0], idxb[0], sem_i[0])

                @pl.when(c * TCHUNK + 1 < SEQ)
                def _():
                    pltpu.async_copy(spm_idx.at[cur, 1], idxb[1], sem_i[1])

                for lt in range(TCHUNK):
                    p = lt % 2

                    @pl.when(c * TCHUNK + lt < SEQ)
                    def _():
                        step(p, lt, c, cur)

                @pl.when(is_prod & (c < n_chunk - 1))
                def _():
                    wait_chunk(c + 1)

                plsc.subcore_barrier()
                return carry

            lax.fori_loop(0, n_chunk, chunk_body, 0)
            pltpu.make_async_copy(slab[0], emb_hbm.at[0, d], sem_o[0]).wait()
            pltpu.make_async_copy(slab[1], emb_hbm.at[0, d], sem_o[1]).wait()

        dd = wid >> 1
        off = pl.multiple_of((wid & 1) * half_w, half_w)

        def piece_src(i):
            return num_hbm.at[i, dd, pl.ds(off, half_w)]

        def piece_dst(i):
            return enc_hbm.at[i, dd, pl.ds(off, half_w)]

        pltpu.async_copy(piece_src(0), numb[0], sem_ni[0])
        pltpu.async_copy(piece_src(1), numb[1], sem_ni[1])

        def nquad(j, c):
            for q in range(4):
                i = 4 * j + q
                b2 = (q + 2) % 4
                pltpu.make_async_copy(
                    num_hbm.at[0, 0, pl.ds(0, half_w)], numb[q], sem_ni[q]
                ).wait()
                pltpu.async_copy(numb[q], piece_dst(i), sem_no[q])

                def drain_and_prefetch():
                    pltpu.make_async_copy(
                        numb[b2], enc_hbm.at[0, 0, pl.ds(0, half_w)], sem_no[b2]
                    ).wait()

                    @pl.when(i + 2 < n_pieces)
                    def _():
                        pltpu.async_copy(piece_src(i + 2), numb[b2], sem_ni[b2])

                if q < 2:
                    @pl.when(j >= 1)
                    def _():
                        drain_and_prefetch()

                    @pl.when(j == 0)
                    def _():
                        pltpu.async_copy(piece_src(i + 2), numb[b2], sem_ni[b2])
                else:
                    drain_and_prefetch()

            return c

        lax.fori_loop(0, n_pieces // 4, nquad, 0)
        pltpu.make_async_copy(numb[2], enc_hbm.at[0, 0, pl.ds(0, half_w)], sem_no[2]).wait()
        pltpu.make_async_copy(numb[3], enc_hbm.at[0, 0, pl.ds(0, half_w)], sem_no[3]).wait()

    return k


def kernel(input_num, input_cat, table):
    B = input_num.shape[0]
    num_t = jnp.transpose(input_num, (1, 2, 0))
    idx_t = jnp.transpose(input_cat.astype(jnp.int32), (1, 0))
    tab_t = jnp.transpose(table, (1, 0))
    enc_t, emb_t = _build_sc_kernel(B, table.shape[0])(num_t, idx_t, tab_t)
    enc = jnp.transpose(enc_t, (2, 0, 1))
    emb = jnp.transpose(emb_t, (2, 0, 1))
    return (enc, emb)


# speedup vs baseline: 14.6440x; 1.2014x over previous
import functools

import jax
import jax.numpy as jnp
from jax import lax
from jax.experimental import pallas as pl
from jax.experimental.pallas import tpu as pltpu
from jax.experimental.pallas import tpu_sc as plsc

ENC = 168
SEQ = 199
D_EMB = 64
D_NUM = 16
LANES = 16
TCHUNK = 8


@functools.cache
def _build_sc_kernel(B, V):
    info = plsc.get_sparse_core_info()
    NC, NS = info.num_cores, info.num_subcores
    NW = NC * NS
    assert D_EMB % NW == 0
    n_phase = D_EMB // NW
    n_chunk = (SEQ + TCHUNK - 1) // TCHUNK
    n_pieces = ENC
    half_w = B // 2

    mesh = plsc.VectorSubcoreMesh(core_axis_name="c", subcore_axis_name="s")

    @functools.partial(
        pl.kernel,
        mesh=mesh,
        out_type=(
            jax.ShapeDtypeStruct((ENC, D_NUM + D_EMB, B), jnp.float32),
            jax.ShapeDtypeStruct((SEQ, D_EMB, B), jnp.float32),
        ),
        scratch_types=[
            pltpu.VMEM((V,), jnp.float32),
            pltpu.VMEM_SHARED((2, TCHUNK, B), jnp.int32),
            [pltpu.VMEM((B,), jnp.int32) for _ in range(2)],
            [pltpu.VMEM((B,), jnp.float32) for _ in range(2)],
            [pltpu.VMEM((B // 2,), jnp.float32) for _ in range(4)],
            [pltpu.SemaphoreType.DMA for _ in range(2)],
            [pltpu.SemaphoreType.DMA for _ in range(2)],
            [pltpu.SemaphoreType.DMA for _ in range(4)],
            [pltpu.SemaphoreType.DMA for _ in range(4)],
            pltpu.SemaphoreType.DMA,
        ],
        compiler_params=pltpu.CompilerParams(needs_layout_passes=False),
    )
    def k(num_hbm, cat_hbm, tab_hbm, enc_hbm, emb_hbm,
          trow_v, spm_idx, idxb, slab, numb, sem_i, sem_o, sem_ni, sem_no,
          sem_p, ):
        sid = lax.axis_index("s")
        wid = sid * NC + lax.axis_index("c")
        is_prod = sid == 0

        def fetch_chunk(c, buf):
            for i in range(TCHUNK):
                @pl.when(c * TCHUNK + i < SEQ)
                def _():
                    pltpu.async_copy(
                        cat_hbm.at[c * TCHUNK + i], spm_idx.at[buf, i], sem_p
                    )

        def wait_chunk(c):
            for i in range(TCHUNK):
                @pl.when(c * TCHUNK + i < SEQ)
                def _():
                    pltpu.make_async_copy(
                        cat_hbm.at[0], spm_idx.at[0, 0], sem_p
                    ).wait()

        def gather_t(ib, sl):
            U = 16

            def g_body(kk, c):
                base = kk * (U * LANES)
                idxs = [ib[pl.ds(base + u * LANES, LANES)] for u in range(U)]
                vals = [plsc.load_gather(trow_v, [iv]) for iv in idxs]
                for u in range(U):
                    sl[pl.ds(base + u * LANES, LANES)] = vals[u]
                return c

            lax.fori_loop(0, B // (U * LANES), g_body, 0)

        dd = wid >> 1
        off = pl.multiple_of((wid & 1) * half_w, half_w)

        def piece_src(i):
            return num_hbm.at[i, dd, pl.ds(off, half_w)]

        def piece_dst(i):
            return enc_hbm.at[i, dd, pl.ds(off, half_w)]

        def num_ring(t, lt):
            q = lt % 4
            b2 = (q + 2) % 4

            @pl.when(t < ENC)
            def _():
                pltpu.make_async_copy(
                    num_hbm.at[0, 0, pl.ds(0, half_w)], numb[q], sem_ni[q]
                ).wait()
                pltpu.async_copy(numb[q], piece_dst(t), sem_no[q])

                @pl.when(t >= 2)
                def _():
                    pltpu.make_async_copy(
                        numb[b2], enc_hbm.at[0, 0, pl.ds(0, half_w)], sem_no[b2]
                    ).wait()

                    @pl.when(t + 2 < n_pieces)
                    def _():
                        pltpu.async_copy(piece_src(t + 2), numb[b2], sem_ni[b2])

                @pl.when(t < 2)
                def _():
                    pltpu.async_copy(piece_src(t + 2), numb[b2], sem_ni[b2])

        for phase in range(n_phase):
            d = phase * NW + wid
            pltpu.sync_copy(tab_hbm.at[d], trow_v)

            @pl.when(is_prod)
            def _():
                fetch_chunk(0, 0)
                wait_chunk(0)

            if phase == 0:
                pltpu.async_copy(piece_src(0), numb[0], sem_ni[0])
                pltpu.async_copy(piece_src(1), numb[1], sem_ni[1])

            plsc.subcore_barrier()

            def step(p, lt, c, cur):
                t = c * TCHUNK + lt
                pltpu.make_async_copy(
                    spm_idx.at[0, 0], idxb[p], sem_i[p]
                ).wait()

                @pl.when((c >= 1) | (lt >= 2))
                def _():
                    pltpu.make_async_copy(
                        slab[p], emb_hbm.at[0, d], sem_o[p]
                    ).wait()

                    @pl.when(t - 2 < ENC)
                    def _():
                        pltpu.make_async_copy(
                            slab[p], enc_hbm.at[0, D_NUM + d], sem_o[p]
                        ).wait()

                gather_t(idxb[p], slab[p])

                if lt + 2 < TCHUNK:
                    @pl.when(t + 2 < SEQ)
                    def _():
                        pltpu.async_copy(
                            spm_idx.at[cur, lt + 2], idxb[p], sem_i[p]
                        )

                pltpu.async_copy(slab[p], emb_hbm.at[t, d], sem_o[p])

                @pl.when(t < ENC)
                def _():
                    pltpu.async_copy(slab[p], enc_hbm.at[t, D_NUM + d], sem_o[p])

                if phase == 0:
                    num_ring(t, lt)

            def chunk_body(c, carry):
                cur = lax.rem(c, 2)
                nxt = 1 - cur

                @pl.when(is_prod & (c < n_chunk - 1))
                def _():
                    fetch_chunk(c + 1, nxt)

                pltpu.async_copy(spm_idx.at[cur, 0], idxb[0], sem_i[0])

                @pl.when(c * TCHUNK + 1 < SEQ)
                def _():
                    pltpu.async_copy(spm_idx.at[cur, 1], idxb[1], sem_i[1])

                for lt in range(TCHUNK):
                    p = lt % 2

                    @pl.when(c * TCHUNK + lt < SEQ)
                    def _():
                        step(p, lt, c, cur)

                @pl.when(is_prod & (c < n_chunk - 1))
                def _():
                    wait_chunk(c + 1)

                plsc.subcore_barrier()
                return carry

            lax.fori_loop(0, n_chunk, chunk_body, 0)
            pltpu.make_async_copy(slab[0], emb_hbm.at[0, d], sem_o[0]).wait()
            pltpu.make_async_copy(slab[1], emb_hbm.at[0, d], sem_o[1]).wait()

            if phase == 0:
                pltpu.make_async_copy(
                    numb[2], enc_hbm.at[0, 0, pl.ds(0, half_w)], sem_no[2]
                ).wait()
                pltpu.make_async_copy(
                    numb[3], enc_hbm.at[0, 0, pl.ds(0, half_w)], sem_no[3]
                ).wait()

    return k


def kernel(input_num, input_cat, table):
    B = input_num.shape[0]
    num_t = jnp.transpose(input_num, (1, 2, 0))
    idx_t = jnp.transpose(input_cat.astype(jnp.int32), (1, 0))
    tab_t = jnp.transpose(table, (1, 0))
    enc_t, emb_t = _build_sc_kernel(B, table.shape[0])(num_t, idx_t, tab_t)
    enc = jnp.transpose(enc_t, (2, 0, 1))
    emb = jnp.transpose(emb_t, (2, 0, 1))
    return (enc, emb)
